# baseline (speedup 1.0000x reference)
import functools

import jax
import jax.numpy as jnp
from jax import lax
from jax.experimental import pallas as pl
from jax.experimental.pallas import tpu as pltpu
from jax.experimental.pallas import tpu_sc as plsc

N_DIM = 64
BATCH = 16384
EPS = 1e-5

L = 16
NC, NS = 2, 16
NW = NC * NS
BPW = BATCH // NW
CHUNK = 128
NCHUNK = BPW // CHUNK
GROUPS = BPW // L
GPC = GROUPS // NCHUNK

LN2 = 0.6931471805599453


def _sc_body(u_hbm, i_hbm, uvect_hbm, ubias_hbm, ivect_hbm, ibias_hbm, gb_hbm,
             out_hbm,
             uidx_v, iidx_v,
             urows_v, irows_v, ubias_v, ibias_v,
             gb_v, out_v, sems):
    wid = lax.axis_index("s") * NC + lax.axis_index("c")
    base = wid * BPW

    pltpu.sync_copy(gb_hbm, gb_v)
    pltpu.sync_copy(u_hbm.at[pl.ds(base, BPW)], uidx_v)
    pltpu.sync_copy(i_hbm.at[pl.ds(base, BPW)], iidx_v)

    def fire(c):
        sl = pl.ds(c * CHUNK, CHUNK)
        b = c % 2
        return [
            pltpu.async_copy(uvect_hbm.at[uidx_v.at[sl]], urows_v.at[b], sems.at[c]),
            pltpu.async_copy(ivect_hbm.at[iidx_v.at[sl]], irows_v.at[b], sems.at[c]),
            pltpu.async_copy(ubias_hbm.at[uidx_v.at[sl]], ubias_v.at[sl], sems.at[c]),
            pltpu.async_copy(ibias_hbm.at[iidx_v.at[sl]], ibias_v.at[sl], sems.at[c]),
        ]

    lane = lax.iota(jnp.int32, L)
    zf = jnp.zeros((L,), jnp.float32)
    gb = gb_v[...]

    def compute_chunk(c):
        b = c % 2
        ubuf = urows_v.at[b]
        ibuf = irows_v.at[b]

        def group_body(gg, _):
            rows = gg * L + lane
            gsl = pl.ds(c * CHUNK + gg * L, L)
            sq0, nu0, nv0 = zf, zf, zf
            sq1, nu1, nv1 = zf, zf, zf
            for d in range(N_DIM):
                dsplat = jnp.full((L,), d, jnp.int32)
                xu = plsc.load_gather(ubuf, [rows, dsplat])
                xi = plsc.load_gather(ibuf, [rows, dsplat])
                diff = xu - xi
                if d % 2 == 0:
                    sq0 = sq0 + diff * diff
                    nu0 = nu0 + xu * xu
                    nv0 = nv0 + xi * xi
                else:
                    sq1 = sq1 + diff * diff
                    nu1 = nu1 + xu * xu
                    nv1 = nv1 + xi * xi
            sq = sq0 + sq1
            nu = nu0 + nu1
            nv = nv0 + nv1
            arg = 1.0 + 2.0 * sq / ((1.0 - nu) * (1.0 - nv) + EPS)
            a = jnp.maximum(arg, 1.0 + EPS)
            x = a * a - 1.0
            yi = 0x5F3759DF - lax.shift_right_logical(plsc.bitcast(x, jnp.int32), 1)
            y = plsc.bitcast(yi, jnp.float32)
            y = y * (1.5 - 0.5 * x * y * y)
            y = y * (1.5 - 0.5 * x * y * y)
            y = y * (1.5 - 0.5 * x * y * y)
            z = a + x * y
            zb = plsc.bitcast(z, jnp.int32)
            e = lax.shift_right_logical(zb, 23) - 127
            m = plsc.bitcast((zb & 0x007FFFFF) | 0x3F800000, jnp.float32)
            t = (m - 1.0) / (m + 1.0)
            t2 = t * t
            lnm = 2.0 * t * (1.0 + t2 * (1.0 / 3.0 + t2 * (0.2 + t2 * (1.0 / 7.0))))
            dist = LN2 * e.astype(jnp.float32) + lnm
            out_v[gsl] = gb + ubias_v[gsl] + ibias_v[gsl] + dist
            return 0

        lax.fori_loop(0, GPC, group_body, 0)

    inflight = {0: fire(0), 1: fire(1)}
    for c in range(NCHUNK):
        for cp in inflight.pop(c):
            cp.wait()
        compute_chunk(c)
        if c + 2 < NCHUNK:
            inflight[c + 2] = fire(c + 2)

    pltpu.sync_copy(out_v, out_hbm.at[pl.ds(base, BPW)])


_sc_kernel = functools.partial(
    pl.kernel,
    out_type=jax.ShapeDtypeStruct((BATCH,), jnp.float32),
    mesh=plsc.VectorSubcoreMesh(core_axis_name="c", subcore_axis_name="s"),
    compiler_params=pltpu.CompilerParams(
        needs_layout_passes=False, use_tc_tiling_on_sc=False
    ),
    scratch_types=[
        pltpu.VMEM((BPW,), jnp.int32),
        pltpu.VMEM((BPW,), jnp.int32),
        pltpu.VMEM((2, CHUNK, N_DIM), jnp.float32),
        pltpu.VMEM((2, CHUNK, N_DIM), jnp.float32),
        pltpu.VMEM((BPW,), jnp.float32),
        pltpu.VMEM((BPW,), jnp.float32),
        pltpu.VMEM((L,), jnp.float32),
        pltpu.VMEM((BPW,), jnp.float32),
        pltpu.SemaphoreType.DMA((NCHUNK,)),
    ],
)(_sc_body)


@jax.jit
def _impl(u, i, user_vect, user_bias, item_vect, item_bias, glob_bias):
    return _sc_kernel(
        u.astype(jnp.int32), i.astype(jnp.int32),
        user_vect, user_bias.reshape(-1),
        item_vect, item_bias.reshape(-1),
        jnp.broadcast_to(glob_bias.reshape(1), (L,)),
    )


def kernel(u, i, user_vect, user_bias, item_vect, item_bias, glob_bias):
    return _impl(u, i, user_vect, user_bias, item_vect, item_bias, glob_bias)
